# baseline (speedup 1.0000x reference)
import functools
import numpy as np
import jax
import jax.numpy as jnp
from jax import lax
from jax.experimental import pallas as pl
from jax.experimental.pallas import tpu as pltpu
from jax.experimental.pallas import tpu_sc as plsc

_CH = 64
_G = 32
_R = 32
_N = 8192
_EPS = 1e-5
_MB = 1024

_OFFS = [(dx, dy, dz) for dx in (-1, 0, 1) for dy in (-1, 0, 1) for dz in (-1, 0, 1)]
_SUB = np.array([[0, 0, 0], [0, 0, 1], [0, 1, 0], [0, 1, 1],
                 [1, 0, 0], [1, 0, 1], [1, 1, 0], [1, 1, 1]], dtype=np.int32)


def _build_maps():
    M1 = np.zeros((27, 8, 27), np.float32)
    M2 = np.zeros((27, 8, 27, 8), np.float32)
    for k, d in enumerate(_OFFS):
        d = np.array(d)
        for s in range(8):
            t = _SUB[s] + d
            e = np.floor_divide(t, 2)
            q = t & 1
            eidx = (e[0] + 1) * 9 + (e[1] + 1) * 3 + (e[2] + 1)
            qidx = q[0] * 4 + q[1] * 2 + q[2]
            M1[k, s, eidx] = 1.0
            M2[k, s, eidx, qidx] = 1.0
    return M1, M2


_M1, _M2 = _build_maps()
_GM64 = np.kron(np.eye(_G, dtype=np.float32),
                np.ones((_CH // _G, _CH // _G), np.float32))
_GM512 = np.tile(_GM64, (8, 8))


def _gn1_body(f_ref, g_ref, b_ref, gm_ref, o_ref):
    f = f_ref[...]
    s1 = jnp.sum(f, axis=0, keepdims=True)
    s2 = jnp.sum(f * f, axis=0, keepdims=True)
    t1 = jnp.dot(s1, gm_ref[...], preferred_element_type=jnp.float32)
    t2 = jnp.dot(s2, gm_ref[...], preferred_element_type=jnp.float32)
    n = 2.0 * f.shape[0]
    mean = t1 / n
    var = t2 / n - mean * mean
    y = (f - mean) * lax.rsqrt(var + _EPS) * g_ref[...] + b_ref[...]
    sy = y * jax.nn.sigmoid(y)
    wide = jnp.concatenate([sy, jnp.zeros_like(sy)], axis=1)
    o_ref[...] = jnp.concatenate(
        [wide, jnp.zeros((1, 2 * _CH), jnp.float32)], axis=0)


def _gn1_silu(feats, gamma, beta):
    return pl.pallas_call(
        _gn1_body,
        out_shape=jax.ShapeDtypeStruct((_N + 1, 2 * _CH), jnp.float32),
    )(feats, gamma.reshape(1, _CH), beta.reshape(1, _CH), jnp.asarray(_GM64))


def _conv1_body(g1_ref, w_ref, b_ref, o_ref, st_ref):
    acc = jnp.dot(g1_ref[...], w_ref[...], preferred_element_type=jnp.float32)
    acc = acc + b_ref[...]
    o_ref[...] = acc
    s1 = jnp.sum(acc, axis=0, keepdims=True)
    s2 = jnp.sum(acc * acc, axis=0, keepdims=True)
    st = jnp.concatenate([s1, s2], axis=0)

    @pl.when(pl.program_id(0) == 0)
    def _():
        st_ref[...] = st

    @pl.when(pl.program_id(0) != 0)
    def _():
        st_ref[...] += st


def _conv1(G1, W1big, b1t):
    grid = (_N // _MB,)
    return pl.pallas_call(
        _conv1_body,
        grid=grid,
        in_specs=[
            pl.BlockSpec((_MB, 27 * 2 * _CH), lambda m: (m, 0)),
            pl.BlockSpec((27 * 2 * _CH, 8 * _CH), lambda m: (0, 0)),
            pl.BlockSpec((1, 8 * _CH), lambda m: (0, 0)),
        ],
        out_specs=[
            pl.BlockSpec((_MB, 8 * _CH), lambda m: (m, 0)),
            pl.BlockSpec((2, 8 * _CH), lambda m: (0, 0)),
        ],
        out_shape=[
            jax.ShapeDtypeStruct((_N, 8 * _CH), jnp.float32),
            jax.ShapeDtypeStruct((2, 8 * _CH), jnp.float32),
        ],
    )(G1, W1big, b1t)


def _gn2_body(x_ref, st_ref, g_ref, b_ref, gm_ref, o_ref):
    t1 = jnp.dot(st_ref[0:1, :], gm_ref[...], preferred_element_type=jnp.float32)
    t2 = jnp.dot(st_ref[1:2, :], gm_ref[...], preferred_element_type=jnp.float32)
    n = 2.0 * 8 * _N
    mean = t1 / n
    var = t2 / n - mean * mean
    x = x_ref[...]
    y = (x - mean) * lax.rsqrt(var + _EPS) * g_ref[...] + b_ref[...]
    o_ref[...] = (y * jax.nn.sigmoid(y)).astype(jnp.bfloat16)


def _gn2_silu(out1p, st, g2t, b2t):
    grid = (_N // _MB,)
    return pl.pallas_call(
        _gn2_body,
        grid=grid,
        in_specs=[
            pl.BlockSpec((_MB, 8 * _CH), lambda m: (m, 0)),
            pl.BlockSpec((2, 8 * _CH), lambda m: (0, 0)),
            pl.BlockSpec((1, 8 * _CH), lambda m: (0, 0)),
            pl.BlockSpec((1, 8 * _CH), lambda m: (0, 0)),
            pl.BlockSpec((8 * _CH, 8 * _CH), lambda m: (0, 0)),
        ],
        out_specs=pl.BlockSpec((_MB, 8 * _CH), lambda m: (m, 0)),
        out_shape=jax.ShapeDtypeStruct((_N, 8 * _CH), jnp.bfloat16),
    )(out1p, st, g2t, b2t, jnp.asarray(_GM512))


def _conv2_body(g2_ref, w_ref, f_ref, b_ref, o_ref):
    k = pl.program_id(1)

    @pl.when(k == 0)
    def _():
        f = f_ref[...]
        o_ref[...] = jnp.concatenate([f] * 8, axis=1) + b_ref[...]

    o_ref[...] += jnp.dot(g2_ref[...], w_ref[...],
                          preferred_element_type=jnp.float32)


def _conv2_skip(G2, W2big, feats, b2t):
    grid = (_N // _MB, 27)
    return pl.pallas_call(
        _conv2_body,
        grid=grid,
        in_specs=[
            pl.BlockSpec((_MB, 8 * _CH), lambda m, k: (m, k)),
            pl.BlockSpec((8 * _CH, 8 * _CH), lambda m, k: (k, 0)),
            pl.BlockSpec((_MB, _CH), lambda m, k: (m, 0)),
            pl.BlockSpec((1, 8 * _CH), lambda m, k: (0, 0)),
        ],
        out_specs=pl.BlockSpec((_MB, 8 * _CH), lambda m, k: (m, 0)),
        out_shape=jax.ShapeDtypeStruct((_N, 8 * _CH), jnp.float32),
    )(G2, W2big, feats, b2t)


_NW = 32
_PPW = _N // _NW
_IPW = _PPW * 27
_CHROWS = 128


def _sc_index(coordsT):
    mesh = plsc.VectorSubcoreMesh(core_axis_name="c", subcore_axis_name="s")

    @functools.partial(
        pl.kernel,
        out_type=jax.ShapeDtypeStruct((_N * 27,), jnp.int32),
        mesh=mesh,
        compiler_params=pltpu.CompilerParams(needs_layout_passes=False),
        scratch_types=[
            pltpu.VMEM((_N,), jnp.int32),
            pltpu.VMEM((_N,), jnp.int32),
            pltpu.VMEM((_N,), jnp.int32),
            pltpu.VMEM((_R ** 3,), jnp.int32),
            pltpu.VMEM((_IPW,), jnp.int32),
        ],
    )
    def idx_kernel(coords_hbm, nbr_hbm, cx_v, cy_v, cz_v, table_v, out_v):
        wid = lax.axis_index("s") * 2 + lax.axis_index("c")
        pltpu.sync_copy(coords_hbm.at[1], cx_v)
        pltpu.sync_copy(coords_hbm.at[2], cy_v)
        pltpu.sync_copy(coords_hbm.at[3], cz_v)
        lanes = lax.iota(jnp.int32, 16)
        neg1 = jnp.full((16,), -1, jnp.int32)

        def init_step(i, _):
            table_v[pl.ds(i * 16, 16)] = neg1
            return 0

        lax.fori_loop(0, _R ** 3 // 16, init_step, 0)

        def scat_step(j, _):
            cx = cx_v[pl.ds(j * 16, 16)]
            cy = cy_v[pl.ds(j * 16, 16)]
            cz = cz_v[pl.ds(j * 16, 16)]
            lin = (cx * _R + cy) * _R + cz
            plsc.store_scatter(table_v, [lin], lanes + j * 16)
            return 0

        lax.fori_loop(0, _N // 16, scat_step, 0)

        def nbr_step(v, _):
            base = wid * _PPW + v * 16
            px = cx_v[pl.ds(base, 16)]
            py = cy_v[pl.ds(base, 16)]
            pz = cz_v[pl.ds(base, 16)]
            locs = lanes + v * 16
            for e, (dx, dy, dz) in enumerate(_OFFS):
                nx, ny, nz = px + dx, py + dy, pz + dz
                valid = ((nx >= 0) & (nx < _R) & (ny >= 0) & (ny < _R)
                         & (nz >= 0) & (nz < _R))
                nlin = jnp.clip((nx * _R + ny) * _R + nz, 0, _R ** 3 - 1)
                idx = plsc.load_gather(table_v, [nlin])
                res = jnp.where(valid & (idx >= 0), idx, _N)
                plsc.store_scatter(out_v, [locs * 27 + e], res)
            return 0

        lax.fori_loop(0, _PPW // 16, nbr_step, 0)
        pltpu.sync_copy(out_v, nbr_hbm.at[pl.ds(wid * _IPW, _IPW)])

    return idx_kernel(coordsT)


def _sc_gather(table, nbr, ncols, dtype):
    mesh = plsc.VectorSubcoreMesh(core_axis_name="c", subcore_axis_name="s")

    @functools.partial(
        pl.kernel,
        out_type=jax.ShapeDtypeStruct((_N * 27, ncols), dtype),
        mesh=mesh,
        compiler_params=pltpu.CompilerParams(needs_layout_passes=False),
        scratch_types=[
            pltpu.VMEM((_IPW,), jnp.int32),
            pltpu.VMEM((_CHROWS, ncols), dtype),
            pltpu.VMEM((_CHROWS, ncols), dtype),
            pltpu.SemaphoreType.DMA,
            pltpu.SemaphoreType.DMA,
        ],
    )
    def gather_kernel(table_hbm, nbr_hbm, out_hbm, idx_v, buf0, buf1,
                      sem0, sem1):
        wid = lax.axis_index("s") * 2 + lax.axis_index("c")
        base = wid * _IPW
        pltpu.sync_copy(nbr_hbm.at[pl.ds(base, _IPW)], idx_v)

        def step(i, _):
            r0 = i * 2 * _CHROWS
            r1 = r0 + _CHROWS
            cp0 = pltpu.async_copy(
                table_hbm.at[idx_v.at[pl.ds(r0, _CHROWS)]], buf0, sem0)
            cp1 = pltpu.async_copy(
                table_hbm.at[idx_v.at[pl.ds(r1, _CHROWS)]], buf1, sem1)
            cp0.wait()
            pltpu.sync_copy(buf0, out_hbm.at[pl.ds(base + r0, _CHROWS), :])
            cp1.wait()
            pltpu.sync_copy(buf1, out_hbm.at[pl.ds(base + r1, _CHROWS), :])
            return 0

        lax.fori_loop(0, _IPW // (2 * _CHROWS), step, 0)

    return gather_kernel(table, nbr)


def kernel(feats, coords, gn1_g, gn1_b, W1, b1, gn2_g, gn2_b, W2, b2):
    W1e = jnp.einsum("kio,kse->eiso", W1, jnp.asarray(_M1))
    W1big = jnp.pad(W1e, ((0, 0), (0, _CH), (0, 0), (0, 0))).reshape(
        27 * 2 * _CH, 8 * _CH)
    W2big = jnp.einsum("kio,kseq->eqiso", W2, jnp.asarray(_M2)).reshape(
        27 * 8 * _CH, 8 * _CH).astype(jnp.bfloat16)
    b1t = jnp.tile(b1, 8).reshape(1, 8 * _CH)
    b2t = jnp.tile(b2, 8).reshape(1, 8 * _CH)
    g2t = jnp.tile(gn2_g, 8).reshape(1, 8 * _CH)
    bt2 = jnp.tile(gn2_b, 8).reshape(1, 8 * _CH)

    nbr = _sc_index(coords.T)

    hp_pad = _gn1_silu(feats, gn1_g, gn1_b)
    G1 = _sc_gather(hp_pad, nbr, 2 * _CH,
                    jnp.float32).reshape(_N, 27 * 2 * _CH)

    out1p, st = _conv1(G1, W1big, b1t)
    in2p = _gn2_silu(out1p, st, g2t, bt2)
    in2p_pad = jnp.concatenate([in2p, jnp.zeros((1, 8 * _CH), in2p.dtype)], 0)
    in2p_i32 = lax.bitcast_convert_type(
        in2p_pad.reshape(_N + 1, 4 * _CH, 2), jnp.int32)
    G2i = _sc_gather(in2p_i32, nbr, 4 * _CH, jnp.int32).reshape(_N, 27 * 4 * _CH)
    G2 = lax.bitcast_convert_type(G2i, jnp.bfloat16).reshape(_N, 27 * 8 * _CH)

    out = _conv2_skip(G2, W2big, feats, b2t)
    h = out.reshape(_N * 8, _CH)

    base = jnp.concatenate([coords[:, :1], coords[:, 1:] * 2], 1)
    add = jnp.concatenate([jnp.zeros((8, 1), jnp.int32), jnp.asarray(_SUB)], 1)
    hc = (base[:, None, :] + add[None, :, :]).reshape(-1, 4)
    return h, hc


# speedup vs baseline: 4.8537x; 4.8537x over previous
import functools
import numpy as np
import jax
import jax.numpy as jnp
from jax import lax
from jax.experimental import pallas as pl
from jax.experimental.pallas import tpu as pltpu
from jax.experimental.pallas import tpu_sc as plsc

_CH = 64
_G = 32
_R = 32
_N = 8192
_EPS = 1e-5
_MB = 1024

_OFFS = [(dx, dy, dz) for dx in (-1, 0, 1) for dy in (-1, 0, 1) for dz in (-1, 0, 1)]
_SUB = np.array([[0, 0, 0], [0, 0, 1], [0, 1, 0], [0, 1, 1],
                 [1, 0, 0], [1, 0, 1], [1, 1, 0], [1, 1, 1]], dtype=np.int32)


def _build_maps():
    M1 = np.zeros((27, 8, 27), np.float32)
    M2 = np.zeros((27, 8, 27, 8), np.float32)
    for k, d in enumerate(_OFFS):
        d = np.array(d)
        for s in range(8):
            t = _SUB[s] + d
            e = np.floor_divide(t, 2)
            q = t & 1
            eidx = (e[0] + 1) * 9 + (e[1] + 1) * 3 + (e[2] + 1)
            qidx = q[0] * 4 + q[1] * 2 + q[2]
            M1[k, s, eidx] = 1.0
            M2[k, s, eidx, qidx] = 1.0
    return M1, M2


_M1, _M2 = _build_maps()
_GM64 = np.kron(np.eye(_G, dtype=np.float32),
                np.ones((_CH // _G, _CH // _G), np.float32))
_GM512 = np.tile(_GM64, (8, 8))


def _gn1_body(f_ref, g_ref, b_ref, gm_ref, o_ref):
    f = f_ref[...]
    s1 = jnp.sum(f, axis=0, keepdims=True)
    s2 = jnp.sum(f * f, axis=0, keepdims=True)
    t1 = jnp.dot(s1, gm_ref[...], preferred_element_type=jnp.float32)
    t2 = jnp.dot(s2, gm_ref[...], preferred_element_type=jnp.float32)
    n = 2.0 * f.shape[0]
    mean = t1 / n
    var = t2 / n - mean * mean
    y = (f - mean) * lax.rsqrt(var + _EPS) * g_ref[...] + b_ref[...]
    sy = y * jax.nn.sigmoid(y)
    wide = jnp.concatenate([sy, jnp.zeros_like(sy)], axis=1)
    o_ref[...] = jnp.concatenate(
        [wide, jnp.zeros((_TROWS - _N, 2 * _CH), jnp.float32)], axis=0)


def _gn1_silu(feats, gamma, beta):
    return pl.pallas_call(
        _gn1_body,
        out_shape=jax.ShapeDtypeStruct((_TROWS, 2 * _CH), jnp.float32),
    )(feats, gamma.reshape(1, _CH), beta.reshape(1, _CH), jnp.asarray(_GM64))


def _conv1_body(g1_ref, w_ref, b_ref, o_ref, st_ref):
    acc = jnp.dot(g1_ref[...], w_ref[...], preferred_element_type=jnp.float32)
    acc = acc + b_ref[...]
    o_ref[...] = acc
    s1 = jnp.sum(acc, axis=0, keepdims=True)
    s2 = jnp.sum(acc * acc, axis=0, keepdims=True)
    st = jnp.concatenate([s1, s2], axis=0)

    @pl.when(pl.program_id(0) == 0)
    def _():
        st_ref[...] = st

    @pl.when(pl.program_id(0) != 0)
    def _():
        st_ref[...] += st


def _conv1(G1, W1big, b1t):
    grid = (_N // _MB,)
    return pl.pallas_call(
        _conv1_body,
        grid=grid,
        in_specs=[
            pl.BlockSpec((_MB, 27 * 2 * _CH), lambda m: (m, 0)),
            pl.BlockSpec((27 * 2 * _CH, 8 * _CH), lambda m: (0, 0)),
            pl.BlockSpec((1, 8 * _CH), lambda m: (0, 0)),
        ],
        out_specs=[
            pl.BlockSpec((_MB, 8 * _CH), lambda m: (m, 0)),
            pl.BlockSpec((2, 8 * _CH), lambda m: (0, 0)),
        ],
        out_shape=[
            jax.ShapeDtypeStruct((_N, 8 * _CH), jnp.float32),
            jax.ShapeDtypeStruct((2, 8 * _CH), jnp.float32),
        ],
    )(G1, W1big, b1t)


def _gn2_body(x_ref, st_ref, g_ref, b_ref, gm_ref, o_ref):
    t1 = jnp.dot(st_ref[0:1, :], gm_ref[...], preferred_element_type=jnp.float32)
    t2 = jnp.dot(st_ref[1:2, :], gm_ref[...], preferred_element_type=jnp.float32)
    n = 2.0 * 8 * _N
    mean = t1 / n
    var = t2 / n - mean * mean
    x = x_ref[...]
    y = (x - mean) * lax.rsqrt(var + _EPS) * g_ref[...] + b_ref[...]
    o_ref[...] = (y * jax.nn.sigmoid(y)).astype(jnp.bfloat16)


def _gn2_silu(out1p, st, g2t, b2t):
    grid = (_N // _MB,)
    return pl.pallas_call(
        _gn2_body,
        grid=grid,
        in_specs=[
            pl.BlockSpec((_MB, 8 * _CH), lambda m: (m, 0)),
            pl.BlockSpec((2, 8 * _CH), lambda m: (0, 0)),
            pl.BlockSpec((1, 8 * _CH), lambda m: (0, 0)),
            pl.BlockSpec((1, 8 * _CH), lambda m: (0, 0)),
            pl.BlockSpec((8 * _CH, 8 * _CH), lambda m: (0, 0)),
        ],
        out_specs=pl.BlockSpec((_MB, 8 * _CH), lambda m: (m, 0)),
        out_shape=jax.ShapeDtypeStruct((_N, 8 * _CH), jnp.bfloat16),
    )(out1p, st, g2t, b2t, jnp.asarray(_GM512))


def _conv2_body(g2_ref, w_ref, f_ref, b_ref, o_ref):
    k = pl.program_id(1)

    @pl.when(k == 0)
    def _():
        f = f_ref[...]
        o_ref[...] = jnp.concatenate([f] * 8, axis=1) + b_ref[...]

    o_ref[...] += jnp.dot(g2_ref[...], w_ref[...],
                          preferred_element_type=jnp.float32)


def _conv2_skip(G2, W2big, feats, b2t):
    grid = (_N // _MB, 27)
    return pl.pallas_call(
        _conv2_body,
        grid=grid,
        in_specs=[
            pl.BlockSpec((_MB, 8 * _CH), lambda m, k: (m, k)),
            pl.BlockSpec((8 * _CH, 8 * _CH), lambda m, k: (k, 0)),
            pl.BlockSpec((_MB, _CH), lambda m, k: (m, 0)),
            pl.BlockSpec((1, 8 * _CH), lambda m, k: (0, 0)),
        ],
        out_specs=pl.BlockSpec((_MB, 8 * _CH), lambda m, k: (m, 0)),
        out_shape=jax.ShapeDtypeStruct((_N, 8 * _CH), jnp.float32),
    )(G2, W2big, feats, b2t)


_NW = 32
_PPW = _N // _NW
_IPW = _PPW * 27
_CHROWS = 128
_TROWS = 8320


def _sc_index(coordsT):
    mesh = plsc.VectorSubcoreMesh(core_axis_name="c", subcore_axis_name="s")

    @functools.partial(
        pl.kernel,
        out_type=jax.ShapeDtypeStruct((_N * 27,), jnp.int32),
        mesh=mesh,
        compiler_params=pltpu.CompilerParams(needs_layout_passes=False),
        scratch_types=[
            pltpu.VMEM((_N,), jnp.int32),
            pltpu.VMEM((_N,), jnp.int32),
            pltpu.VMEM((_N,), jnp.int32),
            pltpu.VMEM((_R ** 3,), jnp.int32),
            pltpu.VMEM((_IPW,), jnp.int32),
        ],
    )
    def idx_kernel(coords_hbm, nbr_hbm, cx_v, cy_v, cz_v, table_v, out_v):
        wid = lax.axis_index("s") * 2 + lax.axis_index("c")
        pltpu.sync_copy(coords_hbm.at[1], cx_v)
        pltpu.sync_copy(coords_hbm.at[2], cy_v)
        pltpu.sync_copy(coords_hbm.at[3], cz_v)
        lanes = lax.iota(jnp.int32, 16)
        neg1 = jnp.full((16,), -1, jnp.int32)

        def init_step(i, _):
            table_v[pl.ds(i * 16, 16)] = neg1
            return 0

        lax.fori_loop(0, _R ** 3 // 16, init_step, 0)

        def scat_step(j, _):
            cx = cx_v[pl.ds(j * 16, 16)]
            cy = cy_v[pl.ds(j * 16, 16)]
            cz = cz_v[pl.ds(j * 16, 16)]
            lin = (cx * _R + cy) * _R + cz
            plsc.store_scatter(table_v, [lin], lanes + j * 16)
            return 0

        lax.fori_loop(0, _N // 16, scat_step, 0)

        def nbr_step(v, _):
            base = wid * _PPW + v * 16
            px = cx_v[pl.ds(base, 16)]
            py = cy_v[pl.ds(base, 16)]
            pz = cz_v[pl.ds(base, 16)]
            locs = lanes + v * 16
            for e, (dx, dy, dz) in enumerate(_OFFS):
                nx, ny, nz = px + dx, py + dy, pz + dz
                valid = ((nx >= 0) & (nx < _R) & (ny >= 0) & (ny < _R)
                         & (nz >= 0) & (nz < _R))
                nlin = jnp.clip((nx * _R + ny) * _R + nz, 0, _R ** 3 - 1)
                idx = plsc.load_gather(table_v, [nlin])
                res = jnp.where(valid & (idx >= 0), idx, _N)
                plsc.store_scatter(out_v, [locs * 27 + e], res)
            return 0

        lax.fori_loop(0, _PPW // 16, nbr_step, 0)
        pltpu.sync_copy(out_v, nbr_hbm.at[pl.ds(wid * _IPW, _IPW)])

    return idx_kernel(coordsT)


def _sc_gather_spmem(table, nbr):
    mesh = plsc.VectorSubcoreMesh(core_axis_name="c", subcore_axis_name="s")

    @functools.partial(
        pl.kernel,
        out_type=jax.ShapeDtypeStruct((_N * 27, 2 * _CH), jnp.float32),
        mesh=mesh,
        compiler_params=pltpu.CompilerParams(needs_layout_passes=False),
        scratch_types=[
            pltpu.VMEM_SHARED((_TROWS, 2 * _CH), jnp.float32),
            pltpu.VMEM((_IPW,), jnp.int32),
            pltpu.VMEM((_CHROWS, 2 * _CH), jnp.float32),
            pltpu.VMEM((_CHROWS, 2 * _CH), jnp.float32),
            pltpu.SemaphoreType.DMA,
            pltpu.SemaphoreType.DMA,
            pltpu.SemaphoreType.DMA,
            pltpu.SemaphoreType.DMA,
        ],
    )
    def gather_kernel(table_hbm, nbr_hbm, out_hbm, shtab, idx_v, buf0, buf1,
                      sg0, sg1, sw0, sw1):
        cid = lax.axis_index("c")
        sid = lax.axis_index("s")
        wid = sid * 2 + cid
        stripe = _TROWS // 16
        pltpu.sync_copy(table_hbm.at[pl.ds(sid * stripe, stripe), :],
                        shtab.at[pl.ds(sid * stripe, stripe), :])
        plsc.subcore_barrier()
        base = wid * _IPW
        pltpu.sync_copy(nbr_hbm.at[pl.ds(base, _IPW)], idx_v)

        def step(i, _):
            r0 = i * 2 * _CHROWS
            r1 = r0 + _CHROWS
            cp0 = pltpu.async_copy(
                shtab.at[idx_v.at[pl.ds(r0, _CHROWS)]], buf0, sg0)
            cp1 = pltpu.async_copy(
                shtab.at[idx_v.at[pl.ds(r1, _CHROWS)]], buf1, sg1)
            cp0.wait()
            wb0 = pltpu.async_copy(
                buf0, out_hbm.at[pl.ds(base + r0, _CHROWS), :], sw0)
            cp1.wait()
            wb1 = pltpu.async_copy(
                buf1, out_hbm.at[pl.ds(base + r1, _CHROWS), :], sw1)
            wb0.wait()
            wb1.wait()
            return 0

        lax.fori_loop(0, _IPW // (2 * _CHROWS), step, 0)

    return gather_kernel(table, nbr)


def _sc_gather_split(table3, nbr):
    mesh = plsc.VectorSubcoreMesh(core_axis_name="c", subcore_axis_name="s")
    rpt = _N * 27 // 16

    @functools.partial(
        pl.kernel,
        out_type=jax.ShapeDtypeStruct((_N * 27, 2, 2 * _CH), jnp.int32),
        mesh=mesh,
        compiler_params=pltpu.CompilerParams(needs_layout_passes=False),
        scratch_types=[
            pltpu.VMEM_SHARED((_TROWS, 2 * _CH), jnp.int32),
            pltpu.VMEM((rpt,), jnp.int32),
            pltpu.VMEM((_CHROWS, 2 * _CH), jnp.int32),
            pltpu.VMEM((_CHROWS, 2 * _CH), jnp.int32),
            pltpu.SemaphoreType.DMA,
            pltpu.SemaphoreType.DMA,
            pltpu.SemaphoreType.DMA,
            pltpu.SemaphoreType.DMA,
        ],
    )
    def gather_kernel(table_hbm, nbr_hbm, out_hbm, shtab, idx_v, buf0, buf1,
                      sg0, sg1, sw0, sw1):
        cid = lax.axis_index("c")
        sid = lax.axis_index("s")
        stripe = _TROWS // 16
        pltpu.sync_copy(table_hbm.at[cid, pl.ds(sid * stripe, stripe), :],
                        shtab.at[pl.ds(sid * stripe, stripe), :])
        plsc.subcore_barrier()
        base = sid * rpt
        pltpu.sync_copy(nbr_hbm.at[pl.ds(base, rpt)], idx_v)

        def step(i, _):
            r0 = i * 2 * _CHROWS
            r1 = r0 + _CHROWS
            cp0 = pltpu.async_copy(
                shtab.at[idx_v.at[pl.ds(r0, _CHROWS)]], buf0, sg0)
            cp1 = pltpu.async_copy(
                shtab.at[idx_v.at[pl.ds(r1, _CHROWS)]], buf1, sg1)
            cp0.wait()
            wb0 = pltpu.async_copy(
                buf0, out_hbm.at[pl.ds(base + r0, _CHROWS), cid, :], sw0)
            cp1.wait()
            wb1 = pltpu.async_copy(
                buf1, out_hbm.at[pl.ds(base + r1, _CHROWS), cid, :], sw1)
            wb0.wait()
            wb1.wait()
            return 0

        lax.fori_loop(0, rpt // (2 * _CHROWS), step, 0)

    return gather_kernel(table3, nbr)


def kernel(feats, coords, gn1_g, gn1_b, W1, b1, gn2_g, gn2_b, W2, b2):
    W1e = jnp.einsum("kio,kse->eiso", W1, jnp.asarray(_M1))
    W1big = jnp.pad(W1e, ((0, 0), (0, _CH), (0, 0), (0, 0))).reshape(
        27 * 2 * _CH, 8 * _CH)
    W2big = jnp.einsum("kio,kseq->eqiso", W2, jnp.asarray(_M2)).reshape(
        27 * 8 * _CH, 8 * _CH).astype(jnp.bfloat16)
    b1t = jnp.tile(b1, 8).reshape(1, 8 * _CH)
    b2t = jnp.tile(b2, 8).reshape(1, 8 * _CH)
    g2t = jnp.tile(gn2_g, 8).reshape(1, 8 * _CH)
    bt2 = jnp.tile(gn2_b, 8).reshape(1, 8 * _CH)

    nbr = _sc_index(coords.T)

    hp_pad = _gn1_silu(feats, gn1_g, gn1_b)
    G1 = _sc_gather_spmem(hp_pad, nbr).reshape(_N, 27 * 2 * _CH)

    out1p, st = _conv1(G1, W1big, b1t)
    in2p = _gn2_silu(out1p, st, g2t, bt2)
    in2p_pad = jnp.concatenate(
        [in2p, jnp.zeros((_TROWS - _N, 8 * _CH), in2p.dtype)], 0)
    in2p_i32 = lax.bitcast_convert_type(
        in2p_pad.reshape(_TROWS, 4 * _CH, 2), jnp.int32)
    tab3 = in2p_i32.reshape(_TROWS, 2, 2 * _CH).transpose(1, 0, 2)
    G2i = _sc_gather_split(tab3, nbr).reshape(_N, 27 * 4 * _CH)
    G2 = lax.bitcast_convert_type(G2i, jnp.bfloat16).reshape(_N, 27 * 8 * _CH)

    out = _conv2_skip(G2, W2big, feats, b2t)
    h = out.reshape(_N * 8, _CH)

    base = jnp.concatenate([coords[:, :1], coords[:, 1:] * 2], 1)
    add = jnp.concatenate([jnp.zeros((8, 1), jnp.int32), jnp.asarray(_SUB)], 1)
    hc = (base[:, None, :] + add[None, :, :]).reshape(-1, 4)
    return h, hc
